# baseline (speedup 1.0000x reference)
import functools
import math

import jax
import jax.numpy as jnp
from jax import lax
from jax.experimental import pallas as pl
from jax.experimental.pallas import tpu as pltpu
from jax.experimental.pallas import tpu_sc as plsc

N = 10000
R = 10240
FEAT = 60
IN_CH = 64
HID = 64
HEADS = 8
HC = HEADS * HID
E0 = 160000
E = E0 + N
EPAD = 172032
NC = 2
NS = 16
NW = NC * NS
EPT = EPAD // NW
EB = 32
NBLK = EPT // EB
ZR = R // NS
BLK = 1024
DUMMY = N


def _leaky(v, s):
    return jnp.maximum(v, s * v)


def _elu(v):
    return jnp.where(v > 0, v, jnp.exp(v) - 1.0)


def _eemb_body(ea_ref, we_ref, o_ref):
    o_ref[...] = ea_ref[...] @ we_ref[...]


def _eemb(eap, We):
    blk = 2048
    return pl.pallas_call(
        _eemb_body,
        grid=(EPAD // blk,),
        in_specs=[
            pl.BlockSpec((blk, 4), lambda i: (i, 0)),
            pl.BlockSpec((4, HC), lambda i: (0, 0)),
        ],
        out_specs=pl.BlockSpec((blk, HC), lambda i: (i, 0)),
        out_shape=jax.ShapeDtypeStruct((EPAD, HC), jnp.float32),
    )(eap, We)


def _proj_block(y, Wl, bl, Wr, br, att, eself):
    xl = y @ Wl + bl
    xr = y @ Wr + br
    m = xl + xr
    if eself is not None:
        m = m + eself
    lk = _leaky(m, 0.2)
    parts = []
    for h in range(HEADS):
        sl = lk[:, h * HID:(h + 1) * HID] * att[h:h + 1, :]
        parts.append(sl.sum(axis=-1, keepdims=True))
    ct = jnp.concatenate(parts + [jnp.zeros_like(parts[0])] * 8, axis=1)
    return xl, xr, ct


def _write_proj(xl, xr, ct, xlf_ref, xrf_ref, xl4_ref, ct_ref):
    xlf_ref[...] = xl
    xrf_ref[...] = xr
    for k in range(4):
        xl4_ref[k] = xl[:, k * 128:(k + 1) * 128]
    ct_ref[...] = ct


def _head_body(y_ref, em_ref, we_ref, wl_ref, bl_ref, wr_ref, br_ref, att_ref,
               xlf_ref, xrf_ref, xl4_ref, ct_ref):
    eself = em_ref[...] @ we_ref[...]
    xl, xr, ct = _proj_block(y_ref[...], wl_ref[...], bl_ref[...], wr_ref[...],
                             br_ref[...], att_ref[...], eself)
    _write_proj(xl, xr, ct, xlf_ref, xrf_ref, xl4_ref, ct_ref)


def _proj_outs():
    return (
        [
            pl.BlockSpec((BLK, HC), lambda i: (i, 0)),
            pl.BlockSpec((BLK, HC), lambda i: (i, 0)),
            pl.BlockSpec((4, BLK, 128), lambda i: (0, i, 0)),
            pl.BlockSpec((BLK, 16), lambda i: (i, 0)),
        ],
        [
            jax.ShapeDtypeStruct((R, HC), jnp.float32),
            jax.ShapeDtypeStruct((R, HC), jnp.float32),
            jax.ShapeDtypeStruct((4, R, 128), jnp.float32),
            jax.ShapeDtypeStruct((R, 16), jnp.float32),
        ],
    )


def _head(y0, emean, We, Wl, bl, Wr, br, att):
    outs, oshapes = _proj_outs()
    rep = lambda *s: pl.BlockSpec(s, lambda i: tuple(0 for _ in s))
    return pl.pallas_call(
        _head_body,
        grid=(R // BLK,),
        in_specs=[
            pl.BlockSpec((BLK, IN_CH), lambda i: (i, 0)),
            rep(1, 4), rep(4, HC), rep(IN_CH, HC), rep(HC,), rep(IN_CH, HC),
            rep(HC,), rep(HEADS, HID),
        ],
        out_specs=outs,
        out_shape=oshapes,
    )(y0, emean, We, Wl, bl, Wr, br, att)


def _gat_merge(den_ref, num_ref, bias, concat):
    den = den_ref[0] + den_ref[1] + 1e-16
    cols = []
    for k in range(4):
        nk = num_ref[0, k] + num_ref[1, k]
        for half in range(2):
            h = 2 * k + half
            cols.append(nk[:, half * 64:(half + 1) * 64] / den[:, h:h + 1])
    if concat:
        y = jnp.concatenate(cols, axis=1) + bias
    else:
        y = sum(cols) * (1.0 / HEADS) + bias
    return _elu(y)


def _mid_body(den_ref, num_ref, bias_ref, wl_ref, bl_ref, wr_ref, br_ref,
              att_ref, xlf_ref, xrf_ref, xl4_ref, ct_ref):
    y = _gat_merge(den_ref, num_ref, bias_ref[...], True)
    xl, xr, ct = _proj_block(y, wl_ref[...], bl_ref[...], wr_ref[...],
                             br_ref[...], att_ref[...], None)
    _write_proj(xl, xr, ct, xlf_ref, xrf_ref, xl4_ref, ct_ref)


def _mid(den, num, bias, Wl, bl, Wr, br, att):
    outs, oshapes = _proj_outs()
    rep = lambda *s: pl.BlockSpec(s, lambda i: tuple(0 for _ in s))
    return pl.pallas_call(
        _mid_body,
        grid=(R // BLK,),
        in_specs=[
            pl.BlockSpec((NC, BLK, 16), lambda i: (0, i, 0)),
            pl.BlockSpec((NC, 4, BLK, 128), lambda i: (0, 0, i, 0)),
            rep(HC,), rep(HC, HC), rep(HC,), rep(HC, HC), rep(HC,),
            rep(HEADS, HID),
        ],
        out_specs=outs,
        out_shape=oshapes,
    )(den, num, bias, Wl, bl, Wr, br, att)


def _tail_body(den_ref, num_ref, g2b_ref, y0_ref, mats_ref, vecs_ref,
               r2w_ref, r2b_ref, o_ref):
    y = _gat_merge(den_ref, num_ref, g2b_ref[...], False)
    y = y + y0_ref[...]
    mem = y
    mats = mats_ref[...]
    vecs = vecs_ref[...]

    def ln(v, gi, bi):
        mu = v.mean(-1, keepdims=True)
        var = ((v - mu) ** 2).mean(-1, keepdims=True)
        return (v - mu) / jnp.sqrt(var + 1e-5) * vecs[gi] + vecs[bi]

    for l in range(2):
        mb = 6 * l
        vb = 12 * l
        sa = (y @ mats[mb + 0] + vecs[vb + 0]) @ mats[mb + 1] + vecs[vb + 1]
        y = ln(y + sa, vb + 6, vb + 7)
        ca = (mem @ mats[mb + 2] + vecs[vb + 2]) @ mats[mb + 3] + vecs[vb + 3]
        y = ln(y + ca, vb + 8, vb + 9)
        ff = jax.nn.relu(y @ mats[mb + 4] + vecs[vb + 4]) @ mats[mb + 5] + vecs[vb + 5]
        y = ln(y + ff, vb + 10, vb + 11)
    y = jnp.tanh(y)
    y = _leaky(y @ mats[12] + vecs[24], 0.01)
    y = _leaky(y @ mats[13] + vecs[25], 0.01)
    o_ref[...] = y @ r2w_ref[...] + r2b_ref[...]


def _tail(den, num, g2b, y0, mats, vecs, r2W, r2b):
    rep = lambda *s: pl.BlockSpec(s, lambda i: tuple(0 for _ in s))
    return pl.pallas_call(
        _tail_body,
        grid=(R // BLK,),
        in_specs=[
            pl.BlockSpec((NC, BLK, 16), lambda i: (0, i, 0)),
            pl.BlockSpec((NC, 4, BLK, 128), lambda i: (0, 0, i, 0)),
            rep(HID,),
            pl.BlockSpec((BLK, HID), lambda i: (i, 0)),
            rep(14, HID, HID), rep(26, HID), rep(HID, 4), rep(1, 4),
        ],
        out_specs=pl.BlockSpec((BLK, 4), lambda i: (i, 0)),
        out_shape=jax.ShapeDtypeStruct((R, 4), jnp.float32),
    )(den, num, g2b, y0, mats, vecs, r2W, r2b)


def _sc_pass1(with_edge):
    mesh = plsc.VectorSubcoreMesh(core_axis_name="c", subcore_axis_name="s")
    buf = lambda shape: [pltpu.VMEM(shape, jnp.float32) for _ in range(2)]
    scratch = [
        buf((EB, HC)),
        buf((EB, HC)),
        buf((EB, HC)),
        buf((EB, 16)),
        buf((EB, 16)),
        pltpu.VMEM((HEADS, HID), jnp.float32),
        pltpu.VMEM((NBLK, EB), jnp.int32),
        pltpu.VMEM((NBLK, EB), jnp.int32),
        pltpu.VMEM_SHARED((R, 16), jnp.float32),
        [pltpu.SemaphoreType.DMA for _ in range(2)],
    ]

    @functools.partial(
        pl.kernel,
        out_type=[
            jax.ShapeDtypeStruct((NC, R, 16), jnp.float32),
            jax.ShapeDtypeStruct((EPAD, 16), jnp.float32),
        ],
        mesh=mesh,
        scratch_types=scratch,
        compiler_params=pltpu.CompilerParams(use_tc_tiling_on_sc=False, needs_layout_passes=False),
    )
    def pass1(src_hbm, dst_hbm, xlf_hbm, xrf_hbm, ct_hbm, att_hbm, em_hbm,
              den_out, ea_out, xj_v, xi_v, em_v, ct_v, ea_v, att_v, si_v,
              di_v, den_sh, gsem):
        cid = lax.axis_index("c")
        sid = lax.axis_index("s")
        wid = sid * NC + cid
        pltpu.sync_copy(att_hbm, att_v)
        pltpu.sync_copy(src_hbm.at[pl.ds(wid * NBLK, NBLK)], si_v)
        pltpu.sync_copy(dst_hbm.at[pl.ds(wid * NBLK, NBLK)], di_v)
        z16 = jnp.zeros((16,), jnp.float32)

        @pl.loop(0, EB)
        def _(r):
            ea_v[0][r] = z16
            ea_v[1][r] = z16

        @pl.loop(0, ZR // EB)
        def _(r):
            pltpu.sync_copy(ea_v[0], den_sh.at[pl.ds(sid * ZR + r * EB, EB)])

        plsc.subcore_barrier()

        lane = lax.iota(jnp.int32, 16)

        def issue(blk, b):
            pltpu.async_copy(xlf_hbm.at[si_v.at[blk]], xj_v[b], gsem[b])
            pltpu.async_copy(xrf_hbm.at[di_v.at[blk]], xi_v[b], gsem[b])
            pltpu.async_copy(ct_hbm.at[di_v.at[blk]], ct_v[b], gsem[b])
            if with_edge:
                pltpu.async_copy(
                    em_hbm.at[pl.ds(wid * EPT + blk * EB, EB)], em_v[b],
                    gsem[b])

        def wait_gather(blk, b):
            pltpu.make_async_copy(xlf_hbm.at[si_v.at[blk]], xj_v[b],
                                  gsem[b]).wait()
            pltpu.make_async_copy(xrf_hbm.at[di_v.at[blk]], xi_v[b],
                                  gsem[b]).wait()
            pltpu.make_async_copy(ct_hbm.at[di_v.at[blk]], ct_v[b],
                                  gsem[b]).wait()
            if with_edge:
                pltpu.make_async_copy(
                    em_hbm.at[pl.ds(wid * EPT + blk * EB, EB)], em_v[b],
                    gsem[b]).wait()

        def compute(blk, b):
            for g in range(EB // 16):
                rowg = lane + (g * 16)

                accs = []
                for hg in range(2):
                    def body(cc, accs4, hg=hg):
                        ccv = jnp.full((16,), 0, jnp.int32) + cc
                        out = []
                        for hh in range(4):
                            h = hg * 4 + hh
                            col = jnp.full((16,), h * HID, jnp.int32) + ccv
                            s = plsc.load_gather(xj_v[b], [rowg, col])
                            s = s + plsc.load_gather(xi_v[b], [rowg, col])
                            if with_edge:
                                s = s + plsc.load_gather(em_v[b], [rowg, col])
                            a = plsc.load_gather(
                                att_v, [jnp.full((16,), h, jnp.int32), ccv])
                            out.append(accs4[hh] + jnp.maximum(s, 0.2 * s) * a)
                        return tuple(out)

                    accs.extend(pl.loop(0, HID, init_carry=tuple([z16] * 4),
                                        unroll=2)(body))
                for h in range(HEADS):
                    hvec = jnp.full((16,), h, jnp.int32)
                    cv = plsc.load_gather(ct_v[b], [rowg, hvec])
                    ea = jnp.exp(accs[h] - cv)
                    plsc.store_scatter(ea_v[b], [rowg, hvec], ea)
            base = wid * EPT + blk * EB
            pltpu.sync_copy(ea_v[b], ea_out.at[pl.ds(base, EB)])
            pltpu.sync_copy(ea_v[b], den_sh.at[di_v.at[blk]], add=True)

        issue(0, 0)

        @pl.loop(0, NBLK, step=2)
        def _(blk):
            wait_gather(blk, 0)
            issue(blk + 1, 1)
            compute(blk, 0)
            wait_gather(blk + 1, 1)

            @pl.when(blk + 2 < NBLK)
            def _():
                issue(blk + 2, 0)

            compute(blk + 1, 1)

        plsc.subcore_barrier()
        pltpu.sync_copy(den_sh.at[pl.ds(sid * ZR, ZR)],
                        den_out.at[cid, pl.ds(sid * ZR, ZR)])

    return pass1


_SC_PASS1_E = _sc_pass1(True)
_SC_PASS1 = _sc_pass1(False)


def _sc_pass2():
    mesh = plsc.VectorSubcoreMesh(core_axis_name="c", subcore_axis_name="s")
    buf = lambda shape, dt: [pltpu.VMEM(shape, dt) for _ in range(2)]
    scratch = [
        buf((EB, 128), jnp.float32),
        buf((EB, 16), jnp.float32),
        buf((EB, 128), jnp.float32),
        pltpu.VMEM((NBLK, EB), jnp.int32),
        pltpu.VMEM((NBLK, EB), jnp.int32),
        pltpu.VMEM_SHARED((R, 128), jnp.float32),
        [pltpu.SemaphoreType.DMA for _ in range(2)],
    ]

    @functools.partial(
        pl.kernel,
        out_type=jax.ShapeDtypeStruct((NC, 4, R, 128), jnp.float32),
        mesh=mesh,
        scratch_types=scratch,
        compiler_params=pltpu.CompilerParams(use_tc_tiling_on_sc=False, needs_layout_passes=False),
    )
    def pass2(src_hbm, dst_hbm, xl4_hbm, ea_hbm, num_out, xj_v, ea_v, w_v,
              si_v, di_v, acc_sh, gsem):
        cid = lax.axis_index("c")
        sid = lax.axis_index("s")
        wid = sid * NC + cid
        pltpu.sync_copy(src_hbm.at[pl.ds(wid * NBLK, NBLK)], si_v)
        pltpu.sync_copy(dst_hbm.at[pl.ds(wid * NBLK, NBLK)], di_v)

        for k in range(4):
            @pl.loop(0, EB)
            def _(r):
                for j in range(8):
                    w_v[0][r, pl.ds(16 * j, 16)] = jnp.zeros(
                        (16,), jnp.float32)

            @pl.loop(0, ZR // EB)
            def _(r):
                pltpu.sync_copy(w_v[0],
                                acc_sh.at[pl.ds(sid * ZR + r * EB, EB)])

            plsc.subcore_barrier()

            def issue(blk, b):
                base = wid * EPT + blk * EB
                pltpu.async_copy(xl4_hbm.at[k].at[si_v.at[blk]], xj_v[b],
                                 gsem[b])
                pltpu.async_copy(ea_hbm.at[pl.ds(base, EB)], ea_v[b],
                                 gsem[b])

            def wait_gather(blk, b):
                base = wid * EPT + blk * EB
                pltpu.make_async_copy(xl4_hbm.at[k].at[si_v.at[blk]],
                                      xj_v[b], gsem[b]).wait()
                pltpu.make_async_copy(ea_hbm.at[pl.ds(base, EB)], ea_v[b],
                                      gsem[b]).wait()

            def compute(blk, b):
                @pl.loop(0, EB)
                def _(e):
                    ev = jnp.full((16,), e, jnp.int32)
                    lo = plsc.load_gather(
                        ea_v[b], [ev, jnp.full((16,), 2 * k, jnp.int32)])
                    hi = plsc.load_gather(
                        ea_v[b], [ev, jnp.full((16,), 2 * k + 1, jnp.int32)])
                    for j in range(8):
                        eav = lo if j < 4 else hi
                        w_v[b][e, pl.ds(16 * j, 16)] = (
                            xj_v[b][e, pl.ds(16 * j, 16)] * eav)

                pltpu.sync_copy(w_v[b], acc_sh.at[di_v.at[blk]], add=True)

            issue(0, 0)

            @pl.loop(0, NBLK, step=2)
            def _(blk):
                wait_gather(blk, 0)
                issue(blk + 1, 1)
                compute(blk, 0)
                wait_gather(blk + 1, 1)

                @pl.when(blk + 2 < NBLK)
                def _():
                    issue(blk + 2, 0)

                compute(blk + 1, 1)

            plsc.subcore_barrier()
            pltpu.sync_copy(acc_sh.at[pl.ds(sid * ZR, ZR)],
                            num_out.at[cid, k, pl.ds(sid * ZR, ZR)])
            plsc.subcore_barrier()

    return pass2


_SC_PASS2 = _sc_pass2()


def kernel(x, edge_index, edge_attr, params, g0_Wl, g0_bl, g0_Wr, g0_br, g0_att, g0_bias, g0_We, g1_Wl, g1_bl, g1_Wr, g1_br, g1_att, g1_bias, g2_Wl, g2_bl, g2_Wr, g2_br, g2_att, g2_bias, t0_sa_Wq, t0_sa_Wk, t0_sa_Wv, t0_sa_Wo, t0_sa_bq, t0_sa_bk, t0_sa_bv, t0_sa_bo, t0_ca_Wq, t0_ca_Wk, t0_ca_Wv, t0_ca_Wo, t0_ca_bq, t0_ca_bk, t0_ca_bv, t0_ca_bo, t0_ff_W1, t0_ff_b1, t0_ff_W2, t0_ff_b2, t0_ln1_g, t0_ln1_b, t0_ln2_g, t0_ln2_b, t0_ln3_g, t0_ln3_b, t1_sa_Wq, t1_sa_Wk, t1_sa_Wv, t1_sa_Wo, t1_sa_bq, t1_sa_bk, t1_sa_bv, t1_sa_bo, t1_ca_Wq, t1_ca_Wk, t1_ca_Wv, t1_ca_Wo, t1_ca_bq, t1_ca_bk, t1_ca_bv, t1_ca_bo, t1_ff_W1, t1_ff_b1, t1_ff_W2, t1_ff_b2, t1_ln1_g, t1_ln1_b, t1_ln2_g, t1_ln2_b, t1_ln3_g, t1_ln3_b, r1_W, r1_b, r11_W, r11_b, r2_W, r2_b):
    B, Nn, _ = x.shape
    y0 = jnp.concatenate(
        [x.reshape(N, FEAT), jnp.broadcast_to(params.reshape(1, -1), (N, 4))],
        axis=1)
    y0p = jnp.pad(y0, ((0, R - N), (0, 0)))
    loops = jnp.arange(N, dtype=edge_index.dtype)
    src = jnp.concatenate([edge_index[0], loops])
    dst = jnp.concatenate([edge_index[1], loops])
    srcp = jnp.pad(src, (0, EPAD - E)).reshape(NW * NBLK, EB)
    dstp = jnp.pad(dst, (0, EPAD - E),
                   constant_values=DUMMY).reshape(NW * NBLK, EB)
    emean = edge_attr.mean(axis=0)
    eap = jnp.concatenate(
        [edge_attr, jnp.broadcast_to(emean[None], (EPAD - E0, 4))], axis=0)

    eemb = _eemb(eap, g0_We)
    xlf, xrf, xl4, ct = _head(y0p, emean.reshape(1, 4), g0_We, g0_Wl, g0_bl,
                              g0_Wr, g0_br, g0_att)
    den, ea = _SC_PASS1_E(srcp, dstp, xlf, xrf, ct, g0_att, eemb)
    num = _SC_PASS2(srcp, dstp, xl4, ea)

    xlf, xrf, xl4, ct = _mid(den, num, g0_bias, g1_Wl, g1_bl, g1_Wr, g1_br,
                             g1_att)
    den, ea = _SC_PASS1(srcp, dstp, xlf, xrf, ct, g1_att, eemb)
    num = _SC_PASS2(srcp, dstp, xl4, ea)

    xlf, xrf, xl4, ct = _mid(den, num, g1_bias, g2_Wl, g2_bl, g2_Wr, g2_br,
                             g2_att)
    den, ea = _SC_PASS1(srcp, dstp, xlf, xrf, ct, g2_att, eemb)
    num = _SC_PASS2(srcp, dstp, xl4, ea)

    mats = jnp.stack([
        t0_sa_Wv, t0_sa_Wo, t0_ca_Wv, t0_ca_Wo, t0_ff_W1, t0_ff_W2,
        t1_sa_Wv, t1_sa_Wo, t1_ca_Wv, t1_ca_Wo, t1_ff_W1, t1_ff_W2,
        r1_W, r11_W,
    ])
    vecs = jnp.stack([
        t0_sa_bv, t0_sa_bo, t0_ca_bv, t0_ca_bo, t0_ff_b1, t0_ff_b2,
        t0_ln1_g, t0_ln1_b, t0_ln2_g, t0_ln2_b, t0_ln3_g, t0_ln3_b,
        t1_sa_bv, t1_sa_bo, t1_ca_bv, t1_ca_bo, t1_ff_b1, t1_ff_b2,
        t1_ln1_g, t1_ln1_b, t1_ln2_g, t1_ln2_b, t1_ln3_g, t1_ln3_b,
        r1_b, r11_b,
    ])
    out = _tail(den, num, g2_bias, y0p, mats, vecs, r2_W, r2_b.reshape(1, 4))
    return out[:N].reshape(B, Nn, 4)


# speedup vs baseline: 2.5805x; 2.5805x over previous
import functools
import math

import jax
import jax.numpy as jnp
from jax import lax
from jax.experimental import pallas as pl
from jax.experimental.pallas import tpu as pltpu
from jax.experimental.pallas import tpu_sc as plsc

N = 10000
R = 10240
FEAT = 60
IN_CH = 64
HID = 64
HEADS = 8
HC = HEADS * HID
E0 = 160000
E = E0 + N
EPAD = 172032
NC = 2
NS = 16
NW = NC * NS
EPT = EPAD // NW
EB = 32
NBLK = EPT // EB
ZR = R // NS
BLK = 1024
DUMMY = N


def _leaky(v, s):
    return jnp.maximum(v, s * v)


def _elu(v):
    return jnp.where(v > 0, v, jnp.exp(v) - 1.0)


def _eemb_body(ea_ref, we_ref, o_ref):
    o_ref[...] = ea_ref[...] @ we_ref[...]


def _eemb(eap, We):
    blk = 2048
    return pl.pallas_call(
        _eemb_body,
        grid=(EPAD // blk,),
        in_specs=[
            pl.BlockSpec((blk, 4), lambda i: (i, 0)),
            pl.BlockSpec((4, HC), lambda i: (0, 0)),
        ],
        out_specs=pl.BlockSpec((blk, HC), lambda i: (i, 0)),
        out_shape=jax.ShapeDtypeStruct((EPAD, HC), jnp.float32),
    )(eap, We)


def _proj_block(y, Wl, bl, Wr, br, att, eself):
    xl = y @ Wl + bl
    xr = y @ Wr + br
    m = xl + xr
    if eself is not None:
        m = m + eself
    lk = _leaky(m, 0.2)
    parts = []
    for h in range(HEADS):
        sl = lk[:, h * HID:(h + 1) * HID] * att[h:h + 1, :]
        parts.append(sl.sum(axis=-1, keepdims=True))
    ct = jnp.concatenate(parts + [jnp.zeros_like(parts[0])] * 8, axis=1)
    return xl, xr, ct


def _write_proj(xl, xr, ct, xlf_ref, xrf_ref, xl4_ref, ct_ref):
    xlf_ref[...] = xl
    xrf_ref[...] = xr
    for k in range(4):
        xl4_ref[k] = xl[:, k * 128:(k + 1) * 128]
    ct_ref[...] = ct


def _head_body(y_ref, em_ref, we_ref, wl_ref, bl_ref, wr_ref, br_ref, att_ref,
               xlf_ref, xrf_ref, xl4_ref, ct_ref):
    eself = em_ref[...] @ we_ref[...]
    xl, xr, ct = _proj_block(y_ref[...], wl_ref[...], bl_ref[...], wr_ref[...],
                             br_ref[...], att_ref[...], eself)
    _write_proj(xl, xr, ct, xlf_ref, xrf_ref, xl4_ref, ct_ref)


def _proj_outs():
    return (
        [
            pl.BlockSpec((BLK, HC), lambda i: (i, 0)),
            pl.BlockSpec((BLK, HC), lambda i: (i, 0)),
            pl.BlockSpec((4, BLK, 128), lambda i: (0, i, 0)),
            pl.BlockSpec((BLK, 16), lambda i: (i, 0)),
        ],
        [
            jax.ShapeDtypeStruct((R, HC), jnp.float32),
            jax.ShapeDtypeStruct((R, HC), jnp.float32),
            jax.ShapeDtypeStruct((4, R, 128), jnp.float32),
            jax.ShapeDtypeStruct((R, 16), jnp.float32),
        ],
    )


def _head(y0, emean, We, Wl, bl, Wr, br, att):
    outs, oshapes = _proj_outs()
    rep = lambda *s: pl.BlockSpec(s, lambda i: tuple(0 for _ in s))
    return pl.pallas_call(
        _head_body,
        grid=(R // BLK,),
        in_specs=[
            pl.BlockSpec((BLK, IN_CH), lambda i: (i, 0)),
            rep(1, 4), rep(4, HC), rep(IN_CH, HC), rep(HC,), rep(IN_CH, HC),
            rep(HC,), rep(HEADS, HID),
        ],
        out_specs=outs,
        out_shape=oshapes,
    )(y0, emean, We, Wl, bl, Wr, br, att)


def _gat_merge(den_ref, num_ref, bias, concat):
    den = den_ref[0] + den_ref[1] + 1e-16
    cols = []
    for k in range(4):
        nk = num_ref[0, k] + num_ref[1, k]
        for half in range(2):
            h = 2 * k + half
            cols.append(nk[:, half * 64:(half + 1) * 64] / den[:, h:h + 1])
    if concat:
        y = jnp.concatenate(cols, axis=1) + bias
    else:
        y = sum(cols) * (1.0 / HEADS) + bias
    return _elu(y)


def _mid_body(den_ref, num_ref, bias_ref, wl_ref, bl_ref, wr_ref, br_ref,
              att_ref, xlf_ref, xrf_ref, xl4_ref, ct_ref):
    y = _gat_merge(den_ref, num_ref, bias_ref[...], True)
    xl, xr, ct = _proj_block(y, wl_ref[...], bl_ref[...], wr_ref[...],
                             br_ref[...], att_ref[...], None)
    _write_proj(xl, xr, ct, xlf_ref, xrf_ref, xl4_ref, ct_ref)


def _mid(den, num, bias, Wl, bl, Wr, br, att):
    outs, oshapes = _proj_outs()
    rep = lambda *s: pl.BlockSpec(s, lambda i: tuple(0 for _ in s))
    return pl.pallas_call(
        _mid_body,
        grid=(R // BLK,),
        in_specs=[
            pl.BlockSpec((NC, BLK, 16), lambda i: (0, i, 0)),
            pl.BlockSpec((NC, 4, BLK, 128), lambda i: (0, 0, i, 0)),
            rep(HC,), rep(HC, HC), rep(HC,), rep(HC, HC), rep(HC,),
            rep(HEADS, HID),
        ],
        out_specs=outs,
        out_shape=oshapes,
    )(den, num, bias, Wl, bl, Wr, br, att)


def _tail_body(den_ref, num_ref, g2b_ref, y0_ref, mats_ref, vecs_ref,
               r2w_ref, r2b_ref, o_ref):
    y = _gat_merge(den_ref, num_ref, g2b_ref[...], False)
    y = y + y0_ref[...]
    mem = y
    mats = mats_ref[...]
    vecs = vecs_ref[...]

    def ln(v, gi, bi):
        mu = v.mean(-1, keepdims=True)
        var = ((v - mu) ** 2).mean(-1, keepdims=True)
        return (v - mu) / jnp.sqrt(var + 1e-5) * vecs[gi] + vecs[bi]

    for l in range(2):
        mb = 6 * l
        vb = 12 * l
        sa = (y @ mats[mb + 0] + vecs[vb + 0]) @ mats[mb + 1] + vecs[vb + 1]
        y = ln(y + sa, vb + 6, vb + 7)
        ca = (mem @ mats[mb + 2] + vecs[vb + 2]) @ mats[mb + 3] + vecs[vb + 3]
        y = ln(y + ca, vb + 8, vb + 9)
        ff = jax.nn.relu(y @ mats[mb + 4] + vecs[vb + 4]) @ mats[mb + 5] + vecs[vb + 5]
        y = ln(y + ff, vb + 10, vb + 11)
    y = jnp.tanh(y)
    y = _leaky(y @ mats[12] + vecs[24], 0.01)
    y = _leaky(y @ mats[13] + vecs[25], 0.01)
    o_ref[...] = y @ r2w_ref[...] + r2b_ref[...]


def _tail(den, num, g2b, y0, mats, vecs, r2W, r2b):
    rep = lambda *s: pl.BlockSpec(s, lambda i: tuple(0 for _ in s))
    return pl.pallas_call(
        _tail_body,
        grid=(R // BLK,),
        in_specs=[
            pl.BlockSpec((NC, BLK, 16), lambda i: (0, i, 0)),
            pl.BlockSpec((NC, 4, BLK, 128), lambda i: (0, 0, i, 0)),
            rep(HID,),
            pl.BlockSpec((BLK, HID), lambda i: (i, 0)),
            rep(14, HID, HID), rep(26, HID), rep(HID, 4), rep(1, 4),
        ],
        out_specs=pl.BlockSpec((BLK, 4), lambda i: (i, 0)),
        out_shape=jax.ShapeDtypeStruct((R, 4), jnp.float32),
    )(den, num, g2b, y0, mats, vecs, r2W, r2b)


def _sc_pass1(with_edge):
    mesh = plsc.VectorSubcoreMesh(core_axis_name="c", subcore_axis_name="s")
    buf = lambda shape: [pltpu.VMEM(shape, jnp.float32) for _ in range(2)]
    scratch = [
        buf((EB, HC)),
        buf((EB, HC)),
        buf((EB, HC)),
        buf((EB, 16)),
        buf((EB, 16)),
        pltpu.VMEM((HEADS, HID), jnp.float32),
        pltpu.VMEM((NBLK, EB), jnp.int32),
        pltpu.VMEM((NBLK, EB), jnp.int32),
        pltpu.VMEM_SHARED((R, 16), jnp.float32),
        [pltpu.SemaphoreType.DMA for _ in range(2)],
    ]

    @functools.partial(
        pl.kernel,
        out_type=[
            jax.ShapeDtypeStruct((NC, R, 16), jnp.float32),
            jax.ShapeDtypeStruct((EPAD, 16), jnp.float32),
        ],
        mesh=mesh,
        scratch_types=scratch,
        compiler_params=pltpu.CompilerParams(use_tc_tiling_on_sc=False, needs_layout_passes=False),
    )
    def pass1(src_hbm, dst_hbm, xlf_hbm, xrf_hbm, ct_hbm, att_hbm, em_hbm,
              den_out, ea_out, xj_v, xi_v, em_v, ct_v, ea_v, att_v, si_v,
              di_v, den_sh, gsem):
        cid = lax.axis_index("c")
        sid = lax.axis_index("s")
        wid = sid * NC + cid
        pltpu.sync_copy(att_hbm, att_v)
        pltpu.sync_copy(src_hbm.at[pl.ds(wid * NBLK, NBLK)], si_v)
        pltpu.sync_copy(dst_hbm.at[pl.ds(wid * NBLK, NBLK)], di_v)
        z16 = jnp.zeros((16,), jnp.float32)

        @pl.loop(0, EB)
        def _(r):
            ea_v[0][r] = z16
            ea_v[1][r] = z16

        @pl.loop(0, ZR // EB)
        def _(r):
            pltpu.sync_copy(ea_v[0], den_sh.at[pl.ds(sid * ZR + r * EB, EB)])

        plsc.subcore_barrier()

        lane = lax.iota(jnp.int32, 16)

        def issue(blk, b):
            pltpu.async_copy(xlf_hbm.at[si_v.at[blk]], xj_v[b], gsem[b])
            pltpu.async_copy(xrf_hbm.at[di_v.at[blk]], xi_v[b], gsem[b])
            pltpu.async_copy(ct_hbm.at[di_v.at[blk]], ct_v[b], gsem[b])
            if with_edge:
                pltpu.async_copy(
                    em_hbm.at[pl.ds(wid * EPT + blk * EB, EB)], em_v[b],
                    gsem[b])

        def wait_gather(blk, b):
            pltpu.make_async_copy(xlf_hbm.at[si_v.at[blk]], xj_v[b],
                                  gsem[b]).wait()
            pltpu.make_async_copy(xrf_hbm.at[di_v.at[blk]], xi_v[b],
                                  gsem[b]).wait()
            pltpu.make_async_copy(ct_hbm.at[di_v.at[blk]], ct_v[b],
                                  gsem[b]).wait()
            if with_edge:
                pltpu.make_async_copy(
                    em_hbm.at[pl.ds(wid * EPT + blk * EB, EB)], em_v[b],
                    gsem[b]).wait()

        def compute(blk, b):
            for g in range(EB // 16):
                rowg = lane + (g * 16)

                accs = []
                for h in range(HEADS):
                    hvec0 = jnp.full((16,), h, jnp.int32)

                    def body(cc, acc, h=h, hvec0=hvec0):
                        dcol = (jnp.full((16,), 0, jnp.int32) + cc
                                + lane) & 63
                        col = jnp.full((16,), h * HID, jnp.int32) + dcol
                        s = plsc.load_gather(xj_v[b], [rowg, col])
                        s = s + plsc.load_gather(xi_v[b], [rowg, col])
                        if with_edge:
                            s = s + plsc.load_gather(em_v[b], [rowg, col])
                        a = plsc.load_gather(att_v, [hvec0, dcol])
                        return acc + jnp.maximum(s, 0.2 * s) * a

                    accs.append(pl.loop(0, HID, init_carry=z16,
                                        unroll=4)(body))
                for h in range(HEADS):
                    hvec = jnp.full((16,), h, jnp.int32)
                    cv = plsc.load_gather(ct_v[b], [rowg, hvec])
                    ea = jnp.exp(accs[h] - cv)
                    plsc.store_scatter(ea_v[b], [rowg, hvec], ea)
            base = wid * EPT + blk * EB
            pltpu.sync_copy(ea_v[b], ea_out.at[pl.ds(base, EB)])
            pltpu.sync_copy(ea_v[b], den_sh.at[di_v.at[blk]], add=True)

        issue(0, 0)

        @pl.loop(0, NBLK, step=2)
        def _(blk):
            wait_gather(blk, 0)
            issue(blk + 1, 1)
            compute(blk, 0)
            wait_gather(blk + 1, 1)

            @pl.when(blk + 2 < NBLK)
            def _():
                issue(blk + 2, 0)

            compute(blk + 1, 1)

        plsc.subcore_barrier()
        pltpu.sync_copy(den_sh.at[pl.ds(sid * ZR, ZR)],
                        den_out.at[cid, pl.ds(sid * ZR, ZR)])

    return pass1


_SC_PASS1_E = _sc_pass1(True)
_SC_PASS1 = _sc_pass1(False)


def _sc_pass2():
    mesh = plsc.VectorSubcoreMesh(core_axis_name="c", subcore_axis_name="s")
    buf = lambda shape, dt: [pltpu.VMEM(shape, dt) for _ in range(2)]
    scratch = [
        buf((EB, 128), jnp.float32),
        buf((EB, 16), jnp.float32),
        buf((EB, 128), jnp.float32),
        pltpu.VMEM((NBLK, EB), jnp.int32),
        pltpu.VMEM((NBLK, EB), jnp.int32),
        pltpu.VMEM_SHARED((R, 128), jnp.float32),
        [pltpu.SemaphoreType.DMA for _ in range(2)],
    ]

    @functools.partial(
        pl.kernel,
        out_type=jax.ShapeDtypeStruct((NC, 4, R, 128), jnp.float32),
        mesh=mesh,
        scratch_types=scratch,
        compiler_params=pltpu.CompilerParams(use_tc_tiling_on_sc=False, needs_layout_passes=False),
    )
    def pass2(src_hbm, dst_hbm, xl4_hbm, ea_hbm, num_out, xj_v, ea_v, w_v,
              si_v, di_v, acc_sh, gsem):
        cid = lax.axis_index("c")
        sid = lax.axis_index("s")
        wid = sid * NC + cid
        pltpu.sync_copy(src_hbm.at[pl.ds(wid * NBLK, NBLK)], si_v)
        pltpu.sync_copy(dst_hbm.at[pl.ds(wid * NBLK, NBLK)], di_v)

        for k in range(4):
            @pl.loop(0, EB)
            def _(r):
                for j in range(8):
                    w_v[0][r, pl.ds(16 * j, 16)] = jnp.zeros(
                        (16,), jnp.float32)

            @pl.loop(0, ZR // EB)
            def _(r):
                pltpu.sync_copy(w_v[0],
                                acc_sh.at[pl.ds(sid * ZR + r * EB, EB)])

            plsc.subcore_barrier()

            def issue(blk, b):
                base = wid * EPT + blk * EB
                pltpu.async_copy(xl4_hbm.at[k].at[si_v.at[blk]], xj_v[b],
                                 gsem[b])
                pltpu.async_copy(ea_hbm.at[pl.ds(base, EB)], ea_v[b],
                                 gsem[b])

            def wait_gather(blk, b):
                base = wid * EPT + blk * EB
                pltpu.make_async_copy(xl4_hbm.at[k].at[si_v.at[blk]],
                                      xj_v[b], gsem[b]).wait()
                pltpu.make_async_copy(ea_hbm.at[pl.ds(base, EB)], ea_v[b],
                                      gsem[b]).wait()

            def compute(blk, b):
                @pl.loop(0, EB)
                def _(e):
                    ev = jnp.full((16,), e, jnp.int32)
                    lo = plsc.load_gather(
                        ea_v[b], [ev, jnp.full((16,), 2 * k, jnp.int32)])
                    hi = plsc.load_gather(
                        ea_v[b], [ev, jnp.full((16,), 2 * k + 1, jnp.int32)])
                    for j in range(8):
                        eav = lo if j < 4 else hi
                        w_v[b][e, pl.ds(16 * j, 16)] = (
                            xj_v[b][e, pl.ds(16 * j, 16)] * eav)

                pltpu.sync_copy(w_v[b], acc_sh.at[di_v.at[blk]], add=True)

            issue(0, 0)

            @pl.loop(0, NBLK, step=2)
            def _(blk):
                wait_gather(blk, 0)
                issue(blk + 1, 1)
                compute(blk, 0)
                wait_gather(blk + 1, 1)

                @pl.when(blk + 2 < NBLK)
                def _():
                    issue(blk + 2, 0)

                compute(blk + 1, 1)

            plsc.subcore_barrier()
            pltpu.sync_copy(acc_sh.at[pl.ds(sid * ZR, ZR)],
                            num_out.at[cid, k, pl.ds(sid * ZR, ZR)])
            plsc.subcore_barrier()

    return pass2


_SC_PASS2 = _sc_pass2()


def kernel(x, edge_index, edge_attr, params, g0_Wl, g0_bl, g0_Wr, g0_br, g0_att, g0_bias, g0_We, g1_Wl, g1_bl, g1_Wr, g1_br, g1_att, g1_bias, g2_Wl, g2_bl, g2_Wr, g2_br, g2_att, g2_bias, t0_sa_Wq, t0_sa_Wk, t0_sa_Wv, t0_sa_Wo, t0_sa_bq, t0_sa_bk, t0_sa_bv, t0_sa_bo, t0_ca_Wq, t0_ca_Wk, t0_ca_Wv, t0_ca_Wo, t0_ca_bq, t0_ca_bk, t0_ca_bv, t0_ca_bo, t0_ff_W1, t0_ff_b1, t0_ff_W2, t0_ff_b2, t0_ln1_g, t0_ln1_b, t0_ln2_g, t0_ln2_b, t0_ln3_g, t0_ln3_b, t1_sa_Wq, t1_sa_Wk, t1_sa_Wv, t1_sa_Wo, t1_sa_bq, t1_sa_bk, t1_sa_bv, t1_sa_bo, t1_ca_Wq, t1_ca_Wk, t1_ca_Wv, t1_ca_Wo, t1_ca_bq, t1_ca_bk, t1_ca_bv, t1_ca_bo, t1_ff_W1, t1_ff_b1, t1_ff_W2, t1_ff_b2, t1_ln1_g, t1_ln1_b, t1_ln2_g, t1_ln2_b, t1_ln3_g, t1_ln3_b, r1_W, r1_b, r11_W, r11_b, r2_W, r2_b):
    B, Nn, _ = x.shape
    y0 = jnp.concatenate(
        [x.reshape(N, FEAT), jnp.broadcast_to(params.reshape(1, -1), (N, 4))],
        axis=1)
    y0p = jnp.pad(y0, ((0, R - N), (0, 0)))
    loops = jnp.arange(N, dtype=edge_index.dtype)
    src = jnp.concatenate([edge_index[0], loops])
    dst = jnp.concatenate([edge_index[1], loops])
    srcp = jnp.pad(src, (0, EPAD - E)).reshape(NW * NBLK, EB)
    dstp = jnp.pad(dst, (0, EPAD - E),
                   constant_values=DUMMY).reshape(NW * NBLK, EB)
    emean = edge_attr.mean(axis=0)
    eap = jnp.concatenate(
        [edge_attr, jnp.broadcast_to(emean[None], (EPAD - E0, 4))], axis=0)

    eemb = _eemb(eap, g0_We)
    xlf, xrf, xl4, ct = _head(y0p, emean.reshape(1, 4), g0_We, g0_Wl, g0_bl,
                              g0_Wr, g0_br, g0_att)
    den, ea = _SC_PASS1_E(srcp, dstp, xlf, xrf, ct, g0_att, eemb)
    num = _SC_PASS2(srcp, dstp, xl4, ea)

    xlf, xrf, xl4, ct = _mid(den, num, g0_bias, g1_Wl, g1_bl, g1_Wr, g1_br,
                             g1_att)
    den, ea = _SC_PASS1(srcp, dstp, xlf, xrf, ct, g1_att, eemb)
    num = _SC_PASS2(srcp, dstp, xl4, ea)

    xlf, xrf, xl4, ct = _mid(den, num, g1_bias, g2_Wl, g2_bl, g2_Wr, g2_br,
                             g2_att)
    den, ea = _SC_PASS1(srcp, dstp, xlf, xrf, ct, g2_att, eemb)
    num = _SC_PASS2(srcp, dstp, xl4, ea)

    mats = jnp.stack([
        t0_sa_Wv, t0_sa_Wo, t0_ca_Wv, t0_ca_Wo, t0_ff_W1, t0_ff_W2,
        t1_sa_Wv, t1_sa_Wo, t1_ca_Wv, t1_ca_Wo, t1_ff_W1, t1_ff_W2,
        r1_W, r11_W,
    ])
    vecs = jnp.stack([
        t0_sa_bv, t0_sa_bo, t0_ca_bv, t0_ca_bo, t0_ff_b1, t0_ff_b2,
        t0_ln1_g, t0_ln1_b, t0_ln2_g, t0_ln2_b, t0_ln3_g, t0_ln3_b,
        t1_sa_bv, t1_sa_bo, t1_ca_bv, t1_ca_bo, t1_ff_b1, t1_ff_b2,
        t1_ln1_g, t1_ln1_b, t1_ln2_g, t1_ln2_b, t1_ln3_g, t1_ln3_b,
        r1_b, r11_b,
    ])
    out = _tail(den, num, g2_bias, y0p, mats, vecs, r2_W, r2_b.reshape(1, 4))
    return out[:N].reshape(B, Nn, 4)
